# baseline (speedup 1.0000x reference)
import functools
import jax
import jax.numpy as jnp
from jax import lax
from jax.experimental import pallas as pl
from jax.experimental.pallas import tpu as pltpu
from jax.experimental.pallas import tpu_sc as plsc

N = 10000
NPAD = 10240
E = 320000
D = 128
H = 128
DE = 3
NC, NS, L = 2, 16, 16
NW = NC * NS
CH = 80
ECH = 4096
NKT = ECH // NW
EP = ECH * CH
ROWS_PER_TILE = NPAD // NS

_mesh = plsc.VectorSubcoreMesh(core_axis_name="c", subcore_axis_name="s",
                               num_cores=NC, num_subcores=NS)
_sc_params = pltpu.CompilerParams(needs_layout_passes=False)


def _bcast_lane(v, e):
    idx = jnp.full((L, 1), e, jnp.int32)
    dnums = lax.GatherDimensionNumbers(offset_dims=(), collapsed_slice_dims=(0,),
                                       start_index_map=(0,))
    return lax.gather(v, idx, dnums, (1,),
                      mode=lax.GatherScatterMode.PROMISE_IN_BOUNDS)


@functools.partial(
    pl.kernel,
    out_type=jax.ShapeDtypeStruct((NC, NPAD, 16), jnp.float32),
    mesh=_mesh,
    compiler_params=_sc_params,
    scratch_types=[
        pltpu.VMEM((CH, 16), jnp.float32),
        pltpu.VMEM((1, CH), jnp.int32),
        pltpu.VMEM_SHARED((NPAD, 16), jnp.float32),
    ],
)
def _sc_deg(ecomb, cnt_hbm, buf_v, idx_v, hist_sh):
    c = lax.axis_index("c")
    s = lax.axis_index("s")
    wid = s * NC + c

    def fill(e, val):
        buf_v[e, :] = jnp.full((16,), val, jnp.float32)

    lax.fori_loop(0, CH, lambda e, _: (fill(e, 0.0), 0)[1], 0)
    for kk in range(ROWS_PER_TILE // CH):
        pltpu.sync_copy(buf_v, hist_sh.at[pl.ds(s * ROWS_PER_TILE + kk * CH, CH)])
    plsc.subcore_barrier()
    lax.fori_loop(0, CH, lambda e, _: (fill(e, 1.0), 0)[1], 0)

    def chunk_body(k, _):
        j = wid + k * NW
        pltpu.sync_copy(ecomb.at[j, pl.ds(0, 1)], idx_v)
        pltpu.sync_copy(buf_v, hist_sh.at[idx_v.at[0]], add=True)
        return 0

    lax.fori_loop(0, NKT, chunk_body, 0)
    plsc.subcore_barrier()
    pltpu.sync_copy(hist_sh.at[pl.ds(s * ROWS_PER_TILE, ROWS_PER_TILE)],
                    cnt_hbm.at[c, pl.ds(s * ROWS_PER_TILE, ROWS_PER_TILE)])


@functools.partial(
    pl.kernel,
    out_type=jax.ShapeDtypeStruct((ECH, 5, CH), jnp.int32),
    mesh=_mesh,
    compiler_params=_sc_params,
    scratch_types=[
        pltpu.VMEM((5, CH), jnp.int32),
        pltpu.VMEM((5, CH), jnp.int32),
        pltpu.VMEM((5, CH), jnp.int32),
        pltpu.VMEM((5, CH), jnp.int32),
        pltpu.VMEM((NPAD,), jnp.float32),
        pltpu.SemaphoreType.DMA,
        pltpu.SemaphoreType.DMA,
        pltpu.SemaphoreType.DMA,
        pltpu.SemaphoreType.DMA,
        pltpu.SemaphoreType.DMA,
        pltpu.SemaphoreType.DMA,
        pltpu.SemaphoreType.DMA,
        pltpu.SemaphoreType.DMA,
    ],
)
def _sc_ea(ecomb, dinv_hbm, out_hbm, eb0, eb1, eb2, eb3, dinv_v,
           sI0, sI1, sI2, sI3, sO0, sO1, sO2, sO3):
    c = lax.axis_index("c")
    s = lax.axis_index("s")
    wid = s * NC + c
    ebs = [eb0, eb1, eb2, eb3]
    semI = [sI0, sI1, sI2, sI3]
    semO = [sO0, sO1, sO2, sO3]

    pltpu.sync_copy(dinv_hbm, dinv_v)

    def idx_issue(kk, sl):
        pltpu.async_copy(ecomb.at[wid + kk * NW], ebs[sl], semI[sl])

    def scale(sl):
        eb = ebs[sl]
        for g in range(CH // 16):
            rowvals = eb[0, pl.ds(g * 16, 16)]
            dvs = plsc.load_gather(dinv_v, [rowvals])
            for kq in range(3):
                a = plsc.bitcast(eb[2 + kq, pl.ds(g * 16, 16)], jnp.float32)
                eb[2 + kq, pl.ds(g * 16, 16)] = plsc.bitcast(dvs * a, jnp.int32)

    def phase(kk, k, early):
        sl = k & 1
        if not early:
            pltpu.make_async_copy(ebs[sl], out_hbm.at[0], semO[sl]).wait()
        pltpu.sync_copy(ecomb.at[wid + kk * NW], ebs[sl])
        scale(sl)
        pltpu.async_copy(ebs[sl], out_hbm.at[wid + kk * NW], semO[sl])

    for k in range(2):
        phase(k, k, early=True)

    def tbody(t, _):
        for k in range(2):
            phase(2 * t + k, k, early=False)
        return 0
    lax.fori_loop(1, NKT // 2, tbody, 0)
    for k in range(2):
        pltpu.make_async_copy(ebs[k], out_hbm.at[0], semO[k]).wait()


@functools.partial(
    pl.kernel,
    out_type=jax.ShapeDtypeStruct((NC, NPAD, H), jnp.float32),
    mesh=_mesh,
    compiler_params=_sc_params,
    scratch_types=[
        pltpu.VMEM((CH, H), jnp.float32),
        pltpu.VMEM((CH, H), jnp.float32),
        pltpu.VMEM((CH, H), jnp.float32),
        pltpu.VMEM((CH, H), jnp.float32),
        pltpu.VMEM((5, CH), jnp.int32),
        pltpu.VMEM((5, CH), jnp.int32),
        pltpu.VMEM((5, CH), jnp.int32),
        pltpu.VMEM((5, CH), jnp.int32),
        pltpu.VMEM((4, CH), jnp.int32),
        pltpu.VMEM((3, H), jnp.float32),
        pltpu.VMEM_SHARED((NPAD, H), jnp.float32),
        pltpu.SemaphoreType.DMA,
        pltpu.SemaphoreType.DMA,
        pltpu.SemaphoreType.DMA,
        pltpu.SemaphoreType.DMA,
        pltpu.SemaphoreType.DMA,
        pltpu.SemaphoreType.DMA,
        pltpu.SemaphoreType.DMA,
        pltpu.SemaphoreType.DMA,
        pltpu.SemaphoreType.DMA,
        pltpu.SemaphoreType.DMA,
        pltpu.SemaphoreType.DMA,
        pltpu.SemaphoreType.DMA,
    ],
)
def _sc_edge(xls_hbm, ecomb, w3_hbm, part_hbm,
             rows0, rows1, rows2, rows3, eb0, eb1, eb2, eb3, scidx_v,
             w3_v, acc_sh,
             sI0, sI1, sI2, sI3, sG0, sG1, sG2, sG3, sS0, sS1, sS2, sS3):
    c = lax.axis_index("c")
    s = lax.axis_index("s")
    wid = s * NC + c
    rows = [rows0, rows1, rows2, rows3]
    ebs = [eb0, eb1, eb2, eb3]
    semI = [sI0, sI1, sI2, sI3]
    semG = [sG0, sG1, sG2, sG3]
    semS = [sS0, sS1, sS2, sS3]

    pltpu.sync_copy(w3_hbm, w3_v)
    wvec = [[w3_v[kk, pl.ds(16 * ss, 16)] for ss in range(8)] for kk in range(3)]

    def zrow(e, _):
        for ss in range(8):
            rows0[e, pl.ds(16 * ss, 16)] = jnp.zeros((16,), jnp.float32)
        return 0
    lax.fori_loop(0, CH, zrow, 0)
    for kk in range(ROWS_PER_TILE // CH):
        pltpu.sync_copy(rows0, acc_sh.at[pl.ds(s * ROWS_PER_TILE + kk * CH, CH)])
    plsc.subcore_barrier()

    def idx_issue(kk, sl):
        pltpu.async_copy(ecomb.at[wid + kk * NW], ebs[sl], semI[sl])

    def idx_wait(sl):
        pltpu.make_async_copy(ecomb.at[0], ebs[sl], semI[sl]).wait()

    def gather_issue(sl):
        pltpu.async_copy(xls_hbm.at[ebs[sl].at[0]], rows[sl], semG[sl])

    def gather_wait(sl):
        pltpu.make_async_copy(xls_hbm.at[ebs[sl].at[0]], rows[sl],
                              semG[sl]).wait()

    def scatter_issue(sl):
        pltpu.async_copy(rows[sl], acc_sh.at[scidx_v.at[sl]], semS[sl],
                         add=True)

    def scatter_wait(sl):
        pltpu.make_async_copy(rows[sl], acc_sh.at[scidx_v.at[sl]],
                              semS[sl]).wait()

    def compute(sl):
        rv = rows[sl]
        eb = ebs[sl]

        def gbody(g, _):
            c0 = plsc.bitcast(eb[2, pl.ds(g * 16, 16)], jnp.float32)
            c1 = plsc.bitcast(eb[3, pl.ds(g * 16, 16)], jnp.float32)
            c2 = plsc.bitcast(eb[4, pl.ds(g * 16, 16)], jnp.float32)

            def ebody(e, _):
                ge = g * 16 + e
                b0 = _bcast_lane(c0, e)
                b1 = _bcast_lane(c1, e)
                b2 = _bcast_lane(c2, e)
                for ss in range(8):
                    z = rv[ge, pl.ds(16 * ss, 16)]
                    z = z + b0 * wvec[0][ss] + b1 * wvec[1][ss] + b2 * wvec[2][ss]
                    rv[ge, pl.ds(16 * ss, 16)] = jnp.maximum(z, 0.0)
                return 0
            lax.fori_loop(0, 16, ebody, 0)
            return 0
        lax.fori_loop(0, CH // 16, gbody, 0)

    def phase(kk, k, early, has2, has4):
        k4 = k & 3
        gather_wait(k4)
        for i in range(CH // 16):
            scidx_v[k4, pl.ds(16 * i, 16)] = ebs[k4][1, pl.ds(16 * i, 16)]
        if has2:
            idx_wait((k + 2) & 3)
            if not early:
                scatter_wait((k + 2) & 3)
            gather_issue((k + 2) & 3)
        compute(k4)
        scatter_issue(k4)
        if has4:
            idx_issue(kk + 4, k4)

    def phase(kk, k, early, has1, has4):
        k4 = k & 3
        if has1:
            idx_wait((k + 1) & 3)
        gather_wait(k4)
        for i in range(CH // 16):
            scidx_v[k4, pl.ds(16 * i, 16)] = ebs[k4][1, pl.ds(16 * i, 16)]
        if not early:
            scatter_wait((k + 2) & 3)
        if has1:
            gather_issue((k + 1) & 3)
        compute(k4)
        scatter_issue(k4)
        if has4:
            idx_issue(kk + 4, k4)

    idx_issue(0, 0)
    idx_issue(1, 1)
    idx_issue(2, 2)
    idx_issue(3, 3)
    idx_wait(0)
    gather_issue(0)
    for k in range(4):
        phase(k, k, early=(k < 2), has1=True, has4=True)

    def tbody(t, _):
        for k in range(4):
            phase(4 * t + k, k, early=False, has1=True, has4=True)
        return 0
    lax.fori_loop(1, NKT // 4 - 1, tbody, 0)

    for k in range(4):
        kk = NKT - 4 + k
        phase(kk, k, early=False, has1=(kk + 1 < NKT), has4=(kk + 4 < NKT))
    scatter_wait(2)
    scatter_wait(3)

    plsc.subcore_barrier()
    pltpu.sync_copy(acc_sh.at[pl.ds(s * ROWS_PER_TILE, ROWS_PER_TILE)],
                    part_hbm.at[c, pl.ds(s * ROWS_PER_TILE, ROWS_PER_TILE)])


def _tc_pre_body(x_ref, w_ref, b_ref, cnt_ref, xls_ref, dinv_ref):
    deg = cnt_ref[0, :, 0:1] + cnt_ref[1, :, 0:1] + 1.0
    dinv = lax.rsqrt(deg)
    dinv_ref[:] = dinv
    xl = jnp.dot(x_ref[:], w_ref[:], preferred_element_type=jnp.float32)
    xls_ref[:] = dinv * (xl + b_ref[:])


def _tc_pre(x_pad, w, b, cnt):
    return pl.pallas_call(
        _tc_pre_body,
        out_shape=(jax.ShapeDtypeStruct((NPAD, H), jnp.float32),
                   jax.ShapeDtypeStruct((NPAD, 1), jnp.float32)),
    )(x_pad, w, b, cnt)


def _bn_relu(p_ref, dinv_ref, gamma_ref, beta_ref):
    h = jnp.maximum(dinv_ref[:] * (p_ref[0] + p_ref[1]), 0.0)
    rmask = (lax.broadcasted_iota(jnp.int32, (NPAD, 1), 0) < N)
    h = jnp.where(rmask, h, 0.0)
    s1 = jnp.sum(h, axis=0, keepdims=True)
    s2 = jnp.sum(h * h, axis=0, keepdims=True)
    mean = s1 / N
    var = s2 / N - mean * mean
    return gamma_ref[:] * (h - mean) * lax.rsqrt(var + 1e-5) + beta_ref[:]


def _tc_mid_body(p_ref, dinv_ref, gamma_ref, beta_ref, w_ref, b_ref, xls_ref):
    hbn = _bn_relu(p_ref, dinv_ref, gamma_ref, beta_ref)
    xl = jnp.dot(hbn, w_ref[:], preferred_element_type=jnp.float32)
    xls_ref[:] = dinv_ref[:] * (xl + b_ref[:])


def _tc_mid(parts, dinv2, gamma, beta, w, b):
    return pl.pallas_call(
        _tc_mid_body,
        out_shape=jax.ShapeDtypeStruct((NPAD, H), jnp.float32),
    )(parts, dinv2, gamma, beta, w, b)


def _tc_final_body(p_ref, dinv_ref, gamma_ref, beta_ref, w_ref, b_ref, out_ref):
    hbn = _bn_relu(p_ref, dinv_ref, gamma_ref, beta_ref)
    out = jnp.dot(hbn, w_ref[:], preferred_element_type=jnp.float32) + b_ref[:]
    out_ref[:] = out[0:N, :]


def _tc_final(parts, dinv2, gamma, beta, w, b):
    return pl.pallas_call(
        _tc_final_body,
        out_shape=jax.ShapeDtypeStruct((N, H), jnp.float32),
    )(parts, dinv2, gamma, beta, w, b)


def kernel(x, edge_index, edge_attr, W1, b1, We1, be1, gamma1, beta1, W2, b2,
           We2, be2, gamma2, beta2, W3, b3, We3, be3, gamma3, beta3, Wl, bl):
    npad_edges = EP - E
    pad_idx = jnp.full((npad_edges,), NPAD - 1, jnp.int32)
    row_p = jnp.concatenate([edge_index[0], pad_idx]).reshape(ECH, 1, CH)
    col_p = jnp.concatenate([edge_index[1], pad_idx]).reshape(ECH, 1, CH)
    attr_p = jnp.concatenate(
        [edge_attr, jnp.zeros((npad_edges, DE), jnp.float32)])
    attr_i = lax.bitcast_convert_type(attr_p, jnp.int32) \
        .reshape(ECH, CH, DE).transpose(0, 2, 1)
    ecomb = jnp.concatenate([row_p, col_p, attr_i], axis=1)
    x_pad = jnp.zeros((NPAD, D), jnp.float32).at[:N].set(x)

    cnt = _sc_deg(ecomb)
    xls, dinv2 = _tc_pre(x_pad, W1, (b1 + be1).reshape(1, H), cnt)
    dinv = dinv2.reshape(NPAD)
    ecombs = _sc_ea(ecomb, dinv)

    parts = _sc_edge(xls, ecombs, We1)
    xls = _tc_mid(parts, dinv2, gamma1.reshape(1, H), beta1.reshape(1, H),
                  W2, (b2 + be2).reshape(1, H))
    parts = _sc_edge(xls, ecombs, We2)
    xls = _tc_mid(parts, dinv2, gamma2.reshape(1, H), beta2.reshape(1, H),
                  W3, (b3 + be3).reshape(1, H))
    parts = _sc_edge(xls, ecombs, We3)
    return _tc_final(parts, dinv2, gamma3.reshape(1, H), beta3.reshape(1, H),
                     Wl, bl.reshape(1, H))


# speedup vs baseline: 1.0056x; 1.0056x over previous
import functools
import jax
import jax.numpy as jnp
from jax import lax
from jax.experimental import pallas as pl
from jax.experimental.pallas import tpu as pltpu
from jax.experimental.pallas import tpu_sc as plsc

N = 10000
NPAD = 10240
E = 320000
D = 128
H = 128
DE = 3
NC, NS, L = 2, 16, 16
NW = NC * NS
CH = 64
ECH = 5120
NKT = ECH // NW
EP = ECH * CH
ROWS_PER_TILE = NPAD // NS

_mesh = plsc.VectorSubcoreMesh(core_axis_name="c", subcore_axis_name="s",
                               num_cores=NC, num_subcores=NS)
_sc_params = pltpu.CompilerParams(needs_layout_passes=False)


def _bcast_lane(v, e):
    idx = jnp.full((L, 1), e, jnp.int32)
    dnums = lax.GatherDimensionNumbers(offset_dims=(), collapsed_slice_dims=(0,),
                                       start_index_map=(0,))
    return lax.gather(v, idx, dnums, (1,),
                      mode=lax.GatherScatterMode.PROMISE_IN_BOUNDS)


@functools.partial(
    pl.kernel,
    out_type=jax.ShapeDtypeStruct((NC, NPAD, 16), jnp.float32),
    mesh=_mesh,
    compiler_params=_sc_params,
    scratch_types=[
        pltpu.VMEM((CH, 16), jnp.float32),
        pltpu.VMEM((1, CH), jnp.int32),
        pltpu.VMEM_SHARED((NPAD, 16), jnp.float32),
    ],
)
def _sc_deg(ecomb, cnt_hbm, buf_v, idx_v, hist_sh):
    c = lax.axis_index("c")
    s = lax.axis_index("s")
    wid = s * NC + c

    def fill(e, val):
        buf_v[e, :] = jnp.full((16,), val, jnp.float32)

    lax.fori_loop(0, CH, lambda e, _: (fill(e, 0.0), 0)[1], 0)
    for kk in range(ROWS_PER_TILE // CH):
        pltpu.sync_copy(buf_v, hist_sh.at[pl.ds(s * ROWS_PER_TILE + kk * CH, CH)])
    plsc.subcore_barrier()
    lax.fori_loop(0, CH, lambda e, _: (fill(e, 1.0), 0)[1], 0)

    def chunk_body(k, _):
        j = wid + k * NW
        pltpu.sync_copy(ecomb.at[j, pl.ds(0, 1)], idx_v)
        pltpu.sync_copy(buf_v, hist_sh.at[idx_v.at[0]], add=True)
        return 0

    lax.fori_loop(0, NKT, chunk_body, 0)
    plsc.subcore_barrier()
    pltpu.sync_copy(hist_sh.at[pl.ds(s * ROWS_PER_TILE, ROWS_PER_TILE)],
                    cnt_hbm.at[c, pl.ds(s * ROWS_PER_TILE, ROWS_PER_TILE)])


@functools.partial(
    pl.kernel,
    out_type=jax.ShapeDtypeStruct((NC, NPAD, H), jnp.float32),
    mesh=_mesh,
    compiler_params=_sc_params,
    scratch_types=[
        pltpu.VMEM((CH, H), jnp.float32),
        pltpu.VMEM((CH, H), jnp.float32),
        pltpu.VMEM((CH, H), jnp.float32),
        pltpu.VMEM((CH, H), jnp.float32),
        pltpu.VMEM((5, CH), jnp.int32),
        pltpu.VMEM((5, CH), jnp.int32),
        pltpu.VMEM((5, CH), jnp.int32),
        pltpu.VMEM((5, CH), jnp.int32),
        pltpu.VMEM((4, CH), jnp.int32),
        pltpu.VMEM((NPAD,), jnp.float32),
        pltpu.VMEM((3, H), jnp.float32),
        pltpu.VMEM_SHARED((NPAD, H), jnp.float32),
        pltpu.SemaphoreType.DMA,
        pltpu.SemaphoreType.DMA,
        pltpu.SemaphoreType.DMA,
        pltpu.SemaphoreType.DMA,
        pltpu.SemaphoreType.DMA,
        pltpu.SemaphoreType.DMA,
        pltpu.SemaphoreType.DMA,
        pltpu.SemaphoreType.DMA,
        pltpu.SemaphoreType.DMA,
        pltpu.SemaphoreType.DMA,
        pltpu.SemaphoreType.DMA,
        pltpu.SemaphoreType.DMA,
    ],
)
def _sc_edge(xls_hbm, ecomb, dinv_hbm, w3_hbm, part_hbm,
             rows0, rows1, rows2, rows3, eb0, eb1, eb2, eb3, scidx_v,
             dinv_v, w3_v, acc_sh,
             sI0, sI1, sI2, sI3, sG0, sG1, sG2, sG3, sS0, sS1, sS2, sS3):
    c = lax.axis_index("c")
    s = lax.axis_index("s")
    wid = s * NC + c
    rows = [rows0, rows1, rows2, rows3]
    ebs = [eb0, eb1, eb2, eb3]
    semI = [sI0, sI1, sI2, sI3]
    semG = [sG0, sG1, sG2, sG3]
    semS = [sS0, sS1, sS2, sS3]

    pltpu.sync_copy(dinv_hbm, dinv_v)
    pltpu.sync_copy(w3_hbm, w3_v)
    wvec = [[w3_v[kk, pl.ds(16 * ss, 16)] for ss in range(8)] for kk in range(3)]

    def zrow(e, _):
        for ss in range(8):
            rows0[e, pl.ds(16 * ss, 16)] = jnp.zeros((16,), jnp.float32)
        return 0
    lax.fori_loop(0, CH, zrow, 0)
    for kk in range(ROWS_PER_TILE // CH):
        pltpu.sync_copy(rows0, acc_sh.at[pl.ds(s * ROWS_PER_TILE + kk * CH, CH)])
    plsc.subcore_barrier()

    def idx_issue(kk, sl):
        pltpu.async_copy(ecomb.at[wid + kk * NW], ebs[sl], semI[sl])

    def idx_wait(sl):
        pltpu.make_async_copy(ecomb.at[0], ebs[sl], semI[sl]).wait()

    def gather_issue(sl):
        pltpu.async_copy(xls_hbm.at[ebs[sl].at[0]], rows[sl], semG[sl])

    def gather_wait(sl):
        pltpu.make_async_copy(xls_hbm.at[ebs[sl].at[0]], rows[sl],
                              semG[sl]).wait()

    def scatter_issue(sl):
        pltpu.async_copy(rows[sl], acc_sh.at[scidx_v.at[sl]], semS[sl],
                         add=True)

    def scatter_wait(sl):
        pltpu.make_async_copy(rows[sl], acc_sh.at[scidx_v.at[sl]],
                              semS[sl]).wait()

    def compute(sl):
        rv = rows[sl]
        eb = ebs[sl]

        def gbody(g, _):
            rowvals = eb[0, pl.ds(g * 16, 16)]
            dvs = plsc.load_gather(dinv_v, [rowvals])
            c0 = dvs * plsc.bitcast(eb[2, pl.ds(g * 16, 16)], jnp.float32)
            c1 = dvs * plsc.bitcast(eb[3, pl.ds(g * 16, 16)], jnp.float32)
            c2 = dvs * plsc.bitcast(eb[4, pl.ds(g * 16, 16)], jnp.float32)

            def ebody(e, _):
                ge = g * 16 + e
                b0 = _bcast_lane(c0, e)
                b1 = _bcast_lane(c1, e)
                b2 = _bcast_lane(c2, e)
                for ss in range(8):
                    z = rv[ge, pl.ds(16 * ss, 16)]
                    z = z + b0 * wvec[0][ss] + b1 * wvec[1][ss] + b2 * wvec[2][ss]
                    rv[ge, pl.ds(16 * ss, 16)] = jnp.maximum(z, 0.0)
                return 0
            lax.fori_loop(0, 16, ebody, 0)
            return 0
        lax.fori_loop(0, CH // 16, gbody, 0)

    def phase(kk, k, early, has2, has4):
        k4 = k & 3
        gather_wait(k4)
        for i in range(CH // 16):
            scidx_v[k4, pl.ds(16 * i, 16)] = ebs[k4][1, pl.ds(16 * i, 16)]
        if has2:
            idx_wait((k + 2) & 3)
            if not early:
                scatter_wait((k + 2) & 3)
            gather_issue((k + 2) & 3)
        compute(k4)
        scatter_issue(k4)
        if has4:
            idx_issue(kk + 4, k4)

    def phase(kk, k, early, has1, has4):
        k4 = k & 3
        if has1:
            idx_wait((k + 1) & 3)
        gather_wait(k4)
        for i in range(CH // 16):
            scidx_v[k4, pl.ds(16 * i, 16)] = ebs[k4][1, pl.ds(16 * i, 16)]
        if not early:
            scatter_wait((k + 2) & 3)
        if has1:
            gather_issue((k + 1) & 3)
        compute(k4)
        scatter_issue(k4)
        if has4:
            idx_issue(kk + 4, k4)

    idx_issue(0, 0)
    idx_issue(1, 1)
    idx_issue(2, 2)
    idx_issue(3, 3)
    idx_wait(0)
    gather_issue(0)
    for k in range(4):
        phase(k, k, early=(k < 2), has1=True, has4=True)

    def tbody(t, _):
        for k in range(4):
            phase(4 * t + k, k, early=False, has1=True, has4=True)
        return 0
    lax.fori_loop(1, NKT // 4 - 1, tbody, 0)

    for k in range(4):
        kk = NKT - 4 + k
        phase(kk, k, early=False, has1=(kk + 1 < NKT), has4=(kk + 4 < NKT))
    scatter_wait(2)
    scatter_wait(3)

    plsc.subcore_barrier()
    pltpu.sync_copy(acc_sh.at[pl.ds(s * ROWS_PER_TILE, ROWS_PER_TILE)],
                    part_hbm.at[c, pl.ds(s * ROWS_PER_TILE, ROWS_PER_TILE)])


def _tc_pre_body(x_ref, w_ref, b_ref, cnt_ref, xls_ref, dinv_ref):
    deg = cnt_ref[0, :, 0:1] + cnt_ref[1, :, 0:1] + 1.0
    dinv = lax.rsqrt(deg)
    dinv_ref[:] = dinv
    xl = jnp.dot(x_ref[:], w_ref[:], preferred_element_type=jnp.float32)
    xls_ref[:] = dinv * (xl + b_ref[:])


def _tc_pre(x_pad, w, b, cnt):
    return pl.pallas_call(
        _tc_pre_body,
        out_shape=(jax.ShapeDtypeStruct((NPAD, H), jnp.float32),
                   jax.ShapeDtypeStruct((NPAD, 1), jnp.float32)),
    )(x_pad, w, b, cnt)


def _bn_relu(p_ref, dinv_ref, gamma_ref, beta_ref):
    h = jnp.maximum(dinv_ref[:] * (p_ref[0] + p_ref[1]), 0.0)
    rmask = (lax.broadcasted_iota(jnp.int32, (NPAD, 1), 0) < N)
    h = jnp.where(rmask, h, 0.0)
    s1 = jnp.sum(h, axis=0, keepdims=True)
    s2 = jnp.sum(h * h, axis=0, keepdims=True)
    mean = s1 / N
    var = s2 / N - mean * mean
    return gamma_ref[:] * (h - mean) * lax.rsqrt(var + 1e-5) + beta_ref[:]


def _tc_mid_body(p_ref, dinv_ref, gamma_ref, beta_ref, w_ref, b_ref, xls_ref):
    hbn = _bn_relu(p_ref, dinv_ref, gamma_ref, beta_ref)
    xl = jnp.dot(hbn, w_ref[:], preferred_element_type=jnp.float32)
    xls_ref[:] = dinv_ref[:] * (xl + b_ref[:])


def _tc_mid(parts, dinv2, gamma, beta, w, b):
    return pl.pallas_call(
        _tc_mid_body,
        out_shape=jax.ShapeDtypeStruct((NPAD, H), jnp.float32),
    )(parts, dinv2, gamma, beta, w, b)


def _tc_final_body(p_ref, dinv_ref, gamma_ref, beta_ref, w_ref, b_ref, out_ref):
    hbn = _bn_relu(p_ref, dinv_ref, gamma_ref, beta_ref)
    out = jnp.dot(hbn, w_ref[:], preferred_element_type=jnp.float32) + b_ref[:]
    out_ref[:] = out[0:N, :]


def _tc_final(parts, dinv2, gamma, beta, w, b):
    return pl.pallas_call(
        _tc_final_body,
        out_shape=jax.ShapeDtypeStruct((N, H), jnp.float32),
    )(parts, dinv2, gamma, beta, w, b)


def kernel(x, edge_index, edge_attr, W1, b1, We1, be1, gamma1, beta1, W2, b2,
           We2, be2, gamma2, beta2, W3, b3, We3, be3, gamma3, beta3, Wl, bl):
    npad_edges = EP - E
    pad_idx = jnp.full((npad_edges,), NPAD - 1, jnp.int32)
    row_p = jnp.concatenate([edge_index[0], pad_idx]).reshape(ECH, 1, CH)
    col_p = jnp.concatenate([edge_index[1], pad_idx]).reshape(ECH, 1, CH)
    attr_p = jnp.concatenate(
        [edge_attr, jnp.zeros((npad_edges, DE), jnp.float32)])
    attr_i = lax.bitcast_convert_type(attr_p, jnp.int32) \
        .reshape(ECH, CH, DE).transpose(0, 2, 1)
    ecomb = jnp.concatenate([row_p, col_p, attr_i], axis=1)
    x_pad = jnp.zeros((NPAD, D), jnp.float32).at[:N].set(x)

    cnt = _sc_deg(ecomb)
    xls, dinv2 = _tc_pre(x_pad, W1, (b1 + be1).reshape(1, H), cnt)
    dinv = dinv2.reshape(NPAD)

    parts = _sc_edge(xls, ecomb, dinv, We1)
    xls = _tc_mid(parts, dinv2, gamma1.reshape(1, H), beta1.reshape(1, H),
                  W2, (b2 + be2).reshape(1, H))
    parts = _sc_edge(xls, ecomb, dinv, We2)
    xls = _tc_mid(parts, dinv2, gamma2.reshape(1, H), beta2.reshape(1, H),
                  W3, (b3 + be3).reshape(1, H))
    parts = _sc_edge(xls, ecomb, dinv, We3)
    return _tc_final(parts, dinv2, gamma3.reshape(1, H), beta3.reshape(1, H),
                     Wl, bl.reshape(1, H))


# speedup vs baseline: 1.0515x; 1.0456x over previous
import functools
import jax
import jax.numpy as jnp
from jax import lax
from jax.experimental import pallas as pl
from jax.experimental.pallas import tpu as pltpu
from jax.experimental.pallas import tpu_sc as plsc

N = 10000
NPAD = 10240
E = 320000
D = 128
H = 128
DE = 3
NC, NS, L = 2, 16, 16
NW = NC * NS
CH = 64
ECH = 5120
NKT = ECH // NW
EP = ECH * CH
ROWS_PER_TILE = NPAD // NS

_mesh = plsc.VectorSubcoreMesh(core_axis_name="c", subcore_axis_name="s",
                               num_cores=NC, num_subcores=NS)
_sc_params = pltpu.CompilerParams(needs_layout_passes=False)


def _bcast_lane(v, e):
    idx = jnp.full((L, 1), e, jnp.int32)
    dnums = lax.GatherDimensionNumbers(offset_dims=(), collapsed_slice_dims=(0,),
                                       start_index_map=(0,))
    return lax.gather(v, idx, dnums, (1,),
                      mode=lax.GatherScatterMode.PROMISE_IN_BOUNDS)


@functools.partial(
    pl.kernel,
    out_type=jax.ShapeDtypeStruct((NC, NPAD, 16), jnp.float32),
    mesh=_mesh,
    compiler_params=_sc_params,
    scratch_types=[
        pltpu.VMEM((CH, 16), jnp.float32),
        pltpu.VMEM((4, 1, CH), jnp.int32),
        pltpu.VMEM((4, CH), jnp.int32),
        pltpu.VMEM_SHARED((NPAD, 16), jnp.float32),
        pltpu.SemaphoreType.DMA,
        pltpu.SemaphoreType.DMA,
        pltpu.SemaphoreType.DMA,
        pltpu.SemaphoreType.DMA,
        pltpu.SemaphoreType.DMA,
        pltpu.SemaphoreType.DMA,
        pltpu.SemaphoreType.DMA,
        pltpu.SemaphoreType.DMA,
    ],
)
def _sc_deg(ecomb, cnt_hbm, buf_v, idx_v, scidx_v, hist_sh,
            sI0, sI1, sI2, sI3, sS0, sS1, sS2, sS3):
    c = lax.axis_index("c")
    s = lax.axis_index("s")
    wid = s * NC + c
    semI = [sI0, sI1, sI2, sI3]
    semS = [sS0, sS1, sS2, sS3]

    def fill(e, val):
        buf_v[e, :] = jnp.full((16,), val, jnp.float32)

    lax.fori_loop(0, CH, lambda e, _: (fill(e, 0.0), 0)[1], 0)
    for kk in range(ROWS_PER_TILE // CH):
        pltpu.sync_copy(buf_v, hist_sh.at[pl.ds(s * ROWS_PER_TILE + kk * CH, CH)])
    plsc.subcore_barrier()
    lax.fori_loop(0, CH, lambda e, _: (fill(e, 1.0), 0)[1], 0)

    def idx_issue(kk, sl):
        pltpu.async_copy(ecomb.at[wid + kk * NW, pl.ds(0, 1)], idx_v.at[sl],
                         semI[sl])

    def scatter_wait(sl):
        pltpu.make_async_copy(buf_v, hist_sh.at[scidx_v.at[sl]],
                              semS[sl]).wait()

    def phase(kk, k, early, has4):
        k4 = k & 3
        pltpu.make_async_copy(ecomb.at[0, pl.ds(0, 1)], idx_v.at[k4],
                              semI[k4]).wait()
        for i in range(CH // 16):
            scidx_v[k4, pl.ds(16 * i, 16)] = idx_v[k4, 0, pl.ds(16 * i, 16)]
        if not early:
            scatter_wait((k + 2) & 3)
        pltpu.async_copy(buf_v, hist_sh.at[scidx_v.at[k4]], semS[k4], add=True)
        if has4:
            idx_issue(kk + 4, k4)

    for k in range(4):
        idx_issue(k, k)
    for k in range(4):
        phase(k, k, early=(k < 2), has4=True)

    def tbody(t, _):
        for k in range(4):
            phase(4 * t + k, k, early=False, has4=True)
        return 0
    lax.fori_loop(1, NKT // 4 - 1, tbody, 0)
    for k in range(4):
        kk = NKT - 4 + k
        phase(kk, k, early=False, has4=(kk + 4 < NKT))
    scatter_wait(2)
    scatter_wait(3)
    plsc.subcore_barrier()
    pltpu.sync_copy(hist_sh.at[pl.ds(s * ROWS_PER_TILE, ROWS_PER_TILE)],
                    cnt_hbm.at[c, pl.ds(s * ROWS_PER_TILE, ROWS_PER_TILE)])


@functools.partial(
    pl.kernel,
    out_type=jax.ShapeDtypeStruct((NC, NPAD, H), jnp.float32),
    mesh=_mesh,
    compiler_params=_sc_params,
    scratch_types=[
        pltpu.VMEM((CH, H), jnp.float32),
        pltpu.VMEM((CH, H), jnp.float32),
        pltpu.VMEM((CH, H), jnp.float32),
        pltpu.VMEM((CH, H), jnp.float32),
        pltpu.VMEM((5, CH), jnp.int32),
        pltpu.VMEM((5, CH), jnp.int32),
        pltpu.VMEM((5, CH), jnp.int32),
        pltpu.VMEM((5, CH), jnp.int32),
        pltpu.VMEM((4, CH), jnp.int32),
        pltpu.VMEM((NPAD,), jnp.float32),
        pltpu.VMEM((3, H), jnp.float32),
        pltpu.VMEM_SHARED((NPAD, H), jnp.float32),
        pltpu.SemaphoreType.DMA,
        pltpu.SemaphoreType.DMA,
        pltpu.SemaphoreType.DMA,
        pltpu.SemaphoreType.DMA,
        pltpu.SemaphoreType.DMA,
        pltpu.SemaphoreType.DMA,
        pltpu.SemaphoreType.DMA,
        pltpu.SemaphoreType.DMA,
        pltpu.SemaphoreType.DMA,
        pltpu.SemaphoreType.DMA,
        pltpu.SemaphoreType.DMA,
        pltpu.SemaphoreType.DMA,
    ],
)
def _sc_edge(xls_hbm, ecomb, dinv_hbm, w3_hbm, part_hbm,
             rows0, rows1, rows2, rows3, eb0, eb1, eb2, eb3, scidx_v,
             dinv_v, w3_v, acc_sh,
             sI0, sI1, sI2, sI3, sG0, sG1, sG2, sG3, sS0, sS1, sS2, sS3):
    c = lax.axis_index("c")
    s = lax.axis_index("s")
    wid = s * NC + c
    rows = [rows0, rows1, rows2, rows3]
    ebs = [eb0, eb1, eb2, eb3]
    semI = [sI0, sI1, sI2, sI3]
    semG = [sG0, sG1, sG2, sG3]
    semS = [sS0, sS1, sS2, sS3]

    pltpu.sync_copy(dinv_hbm, dinv_v)
    pltpu.sync_copy(w3_hbm, w3_v)
    wvec = [[w3_v[kk, pl.ds(16 * ss, 16)] for ss in range(8)] for kk in range(3)]

    def zrow(e, _):
        for ss in range(8):
            rows0[e, pl.ds(16 * ss, 16)] = jnp.zeros((16,), jnp.float32)
        return 0
    lax.fori_loop(0, CH, zrow, 0)
    for kk in range(ROWS_PER_TILE // CH):
        pltpu.sync_copy(rows0, acc_sh.at[pl.ds(s * ROWS_PER_TILE + kk * CH, CH)])
    plsc.subcore_barrier()

    def idx_issue(kk, sl):
        pltpu.async_copy(ecomb.at[wid + kk * NW], ebs[sl], semI[sl])

    def idx_wait(sl):
        pltpu.make_async_copy(ecomb.at[0], ebs[sl], semI[sl]).wait()

    def gather_issue(sl):
        pltpu.async_copy(xls_hbm.at[ebs[sl].at[0]], rows[sl], semG[sl])

    def gather_wait(sl):
        pltpu.make_async_copy(xls_hbm.at[ebs[sl].at[0]], rows[sl],
                              semG[sl]).wait()

    def scatter_issue(sl):
        pltpu.async_copy(rows[sl], acc_sh.at[scidx_v.at[sl]], semS[sl],
                         add=True)

    def scatter_wait(sl):
        pltpu.make_async_copy(rows[sl], acc_sh.at[scidx_v.at[sl]],
                              semS[sl]).wait()

    def compute(sl):
        rv = rows[sl]
        eb = ebs[sl]

        def gbody(g, _):
            rowvals = eb[0, pl.ds(g * 16, 16)]
            dvs = plsc.load_gather(dinv_v, [rowvals])
            c0 = dvs * plsc.bitcast(eb[2, pl.ds(g * 16, 16)], jnp.float32)
            c1 = dvs * plsc.bitcast(eb[3, pl.ds(g * 16, 16)], jnp.float32)
            c2 = dvs * plsc.bitcast(eb[4, pl.ds(g * 16, 16)], jnp.float32)

            def ebody(e, _):
                ge = g * 16 + e
                b0 = _bcast_lane(c0, e)
                b1 = _bcast_lane(c1, e)
                b2 = _bcast_lane(c2, e)
                for ss in range(8):
                    z = rv[ge, pl.ds(16 * ss, 16)]
                    z = z + b0 * wvec[0][ss] + b1 * wvec[1][ss] + b2 * wvec[2][ss]
                    rv[ge, pl.ds(16 * ss, 16)] = jnp.maximum(z, 0.0)
                return 0
            lax.fori_loop(0, 16, ebody, 0)
            return 0
        lax.fori_loop(0, CH // 16, gbody, 0)

    def phase(kk, k, early, has2, has4):
        k4 = k & 3
        gather_wait(k4)
        for i in range(CH // 16):
            scidx_v[k4, pl.ds(16 * i, 16)] = ebs[k4][1, pl.ds(16 * i, 16)]
        if has2:
            idx_wait((k + 2) & 3)
            if not early:
                scatter_wait((k + 2) & 3)
            gather_issue((k + 2) & 3)
        compute(k4)
        scatter_issue(k4)
        if has4:
            idx_issue(kk + 4, k4)

    def phase(kk, k, early, has1, has4):
        k4 = k & 3
        if has1:
            idx_wait((k + 1) & 3)
        gather_wait(k4)
        for i in range(CH // 16):
            scidx_v[k4, pl.ds(16 * i, 16)] = ebs[k4][1, pl.ds(16 * i, 16)]
        if not early:
            scatter_wait((k + 2) & 3)
        if has1:
            gather_issue((k + 1) & 3)
        compute(k4)
        scatter_issue(k4)
        if has4:
            idx_issue(kk + 4, k4)

    idx_issue(0, 0)
    idx_issue(1, 1)
    idx_issue(2, 2)
    idx_issue(3, 3)
    idx_wait(0)
    gather_issue(0)
    for k in range(4):
        phase(k, k, early=(k < 2), has1=True, has4=True)

    def tbody(t, _):
        for k in range(4):
            phase(4 * t + k, k, early=False, has1=True, has4=True)
        return 0
    lax.fori_loop(1, NKT // 4 - 1, tbody, 0)

    for k in range(4):
        kk = NKT - 4 + k
        phase(kk, k, early=False, has1=(kk + 1 < NKT), has4=(kk + 4 < NKT))
    scatter_wait(2)
    scatter_wait(3)

    plsc.subcore_barrier()
    pltpu.sync_copy(acc_sh.at[pl.ds(s * ROWS_PER_TILE, ROWS_PER_TILE)],
                    part_hbm.at[c, pl.ds(s * ROWS_PER_TILE, ROWS_PER_TILE)])


def _tc_pre_body(x_ref, w_ref, b_ref, cnt_ref, xls_ref, dinv_ref):
    deg = cnt_ref[0, :, 0:1] + cnt_ref[1, :, 0:1] + 1.0
    dinv = lax.rsqrt(deg)
    dinv_ref[:] = dinv
    xl = jnp.dot(x_ref[:], w_ref[:], preferred_element_type=jnp.float32)
    xls_ref[:] = dinv * (xl + b_ref[:])


def _tc_pre(x_pad, w, b, cnt):
    return pl.pallas_call(
        _tc_pre_body,
        out_shape=(jax.ShapeDtypeStruct((NPAD, H), jnp.float32),
                   jax.ShapeDtypeStruct((NPAD, 1), jnp.float32)),
    )(x_pad, w, b, cnt)


def _bn_relu(p_ref, dinv_ref, gamma_ref, beta_ref):
    h = jnp.maximum(dinv_ref[:] * (p_ref[0] + p_ref[1]), 0.0)
    rmask = (lax.broadcasted_iota(jnp.int32, (NPAD, 1), 0) < N)
    h = jnp.where(rmask, h, 0.0)
    s1 = jnp.sum(h, axis=0, keepdims=True)
    s2 = jnp.sum(h * h, axis=0, keepdims=True)
    mean = s1 / N
    var = s2 / N - mean * mean
    return gamma_ref[:] * (h - mean) * lax.rsqrt(var + 1e-5) + beta_ref[:]


def _tc_mid_body(p_ref, dinv_ref, gamma_ref, beta_ref, w_ref, b_ref, xls_ref):
    hbn = _bn_relu(p_ref, dinv_ref, gamma_ref, beta_ref)
    xl = jnp.dot(hbn, w_ref[:], preferred_element_type=jnp.float32)
    xls_ref[:] = dinv_ref[:] * (xl + b_ref[:])


def _tc_mid(parts, dinv2, gamma, beta, w, b):
    return pl.pallas_call(
        _tc_mid_body,
        out_shape=jax.ShapeDtypeStruct((NPAD, H), jnp.float32),
    )(parts, dinv2, gamma, beta, w, b)


def _tc_final_body(p_ref, dinv_ref, gamma_ref, beta_ref, w_ref, b_ref, out_ref):
    hbn = _bn_relu(p_ref, dinv_ref, gamma_ref, beta_ref)
    out = jnp.dot(hbn, w_ref[:], preferred_element_type=jnp.float32) + b_ref[:]
    out_ref[:] = out[0:N, :]


def _tc_final(parts, dinv2, gamma, beta, w, b):
    return pl.pallas_call(
        _tc_final_body,
        out_shape=jax.ShapeDtypeStruct((N, H), jnp.float32),
    )(parts, dinv2, gamma, beta, w, b)


def kernel(x, edge_index, edge_attr, W1, b1, We1, be1, gamma1, beta1, W2, b2,
           We2, be2, gamma2, beta2, W3, b3, We3, be3, gamma3, beta3, Wl, bl):
    npad_edges = EP - E
    pad_idx = jnp.full((npad_edges,), NPAD - 1, jnp.int32)
    row_p = jnp.concatenate([edge_index[0], pad_idx]).reshape(ECH, 1, CH)
    col_p = jnp.concatenate([edge_index[1], pad_idx]).reshape(ECH, 1, CH)
    attr_p = jnp.concatenate(
        [edge_attr, jnp.zeros((npad_edges, DE), jnp.float32)])
    attr_i = lax.bitcast_convert_type(attr_p, jnp.int32) \
        .reshape(ECH, CH, DE).transpose(0, 2, 1)
    ecomb = jnp.concatenate([row_p, col_p, attr_i], axis=1)
    x_pad = jnp.zeros((NPAD, D), jnp.float32).at[:N].set(x)

    cnt = _sc_deg(ecomb)
    xls, dinv2 = _tc_pre(x_pad, W1, (b1 + be1).reshape(1, H), cnt)
    dinv = dinv2.reshape(NPAD)

    parts = _sc_edge(xls, ecomb, dinv, We1)
    xls = _tc_mid(parts, dinv2, gamma1.reshape(1, H), beta1.reshape(1, H),
                  W2, (b2 + be2).reshape(1, H))
    parts = _sc_edge(xls, ecomb, dinv, We2)
    xls = _tc_mid(parts, dinv2, gamma2.reshape(1, H), beta2.reshape(1, H),
                  W3, (b3 + be3).reshape(1, H))
    parts = _sc_edge(xls, ecomb, dinv, We3)
    return _tc_final(parts, dinv2, gamma3.reshape(1, H), beta3.reshape(1, H),
                     Wl, bl.reshape(1, H))
